# Initial kernel scaffold; baseline (speedup 1.0000x reference)
import jax
import jax.numpy as jnp
from jax.experimental import pallas as pl


def kernel(x, batch):
    raise NotImplementedError("write your pallas kernel here")


# speedup vs baseline: 2.3023x; 2.3023x over previous
import functools

import jax
import jax.numpy as jnp
from jax import lax
from jax.experimental import pallas as pl
from jax.experimental.pallas import tpu as pltpu
from jax.experimental.pallas import tpu_sc as plsc

N_NODES = 100000
D = 128
NSEG = 128
NC, NS = 2, 16
NW = NC * NS
CHUNK = 3128
BLK = 128
NB = -(-CHUNK // BLK)
LAST_START = N_NODES - BLK
NEG_INF = float("-inf")


def _sc_partial_max(x, batch_i32):
    mesh = plsc.VectorSubcoreMesh(
        core_axis_name="c", subcore_axis_name="s",
        num_cores=NC, num_subcores=NS)

    @functools.partial(
        pl.kernel,
        out_type=jax.ShapeDtypeStruct((NW, NSEG, D), jnp.float32),
        mesh=mesh,
        scratch_types=[
            pltpu.VMEM((BLK,), jnp.int32),
            pltpu.VMEM((BLK, D), jnp.float32),
            pltpu.VMEM((NSEG, D), jnp.float32),
            pltpu.SemaphoreType.DMA,
            pltpu.SemaphoreType.DMA,
        ],
    )
    def k(x_hbm, b_hbm, part_hbm, ids_v, buf_v, acc_v, sem_i, sem_x):
        wid = lax.axis_index("s") * NC + lax.axis_index("c")
        base = wid * CHUNK

        neg = jnp.full((16,), NEG_INF, jnp.float32)

        def init_body(i, c):
            for f in range(D // 16):
                acc_v[i, pl.ds(16 * f, 16)] = neg
            return c
        lax.fori_loop(0, NSEG, init_body, 0)

        def blk_body(g, c):
            start = jnp.minimum(base + g * BLK, LAST_START)
            cp_i = pltpu.async_copy(b_hbm.at[pl.ds(start, BLK)], ids_v, sem_i)
            cp_x = pltpu.async_copy(x_hbm.at[pl.ds(start, BLK)], buf_v, sem_x)
            cp_i.wait()
            cp_x.wait()

            def grp_body(t, rc):
                idv = ids_v[pl.ds(t * 16, 16)]
                for j in range(16):
                    seg = idv[j]
                    r = t * 16 + j
                    for f in range(D // 16):
                        sl = pl.ds(16 * f, 16)
                        acc_v[seg, sl] = jnp.maximum(acc_v[seg, sl],
                                                     buf_v[r, sl])
                return rc
            lax.fori_loop(0, BLK // 16, grp_body, 0)
            return c
        lax.fori_loop(0, NB, blk_body, 0)

        pltpu.sync_copy(acc_v, part_hbm.at[wid])

    return k(x, batch_i32)


def _tc_combine(part):
    def body(p_ref, o_ref):
        o_ref[...] = jnp.max(p_ref[...], axis=0)

    return pl.pallas_call(
        body,
        out_shape=jax.ShapeDtypeStruct((NSEG, D), jnp.float32),
    )(part)


def kernel(x, batch):
    part = _sc_partial_max(x, batch.astype(jnp.int32))
    return _tc_combine(part)


# speedup vs baseline: 5.2031x; 2.2599x over previous
import functools

import jax
import jax.numpy as jnp
from jax import lax
from jax.experimental import pallas as pl
from jax.experimental.pallas import tpu as pltpu
from jax.experimental.pallas import tpu_sc as plsc

N_NODES = 100000
D = 128
NF = D // 16
NSEG = 128
NC, NS = 2, 16
NW = NC * NS
CHUNK = 3128
BLK = 128
NBLK = 26
LAST_START = N_NODES - BLK
NEG_INF = float("-inf")


def _sc_partial_max(x, batch_i32):
    mesh = plsc.VectorSubcoreMesh(
        core_axis_name="c", subcore_axis_name="s",
        num_cores=NC, num_subcores=NS)

    @functools.partial(
        pl.kernel,
        out_type=jax.ShapeDtypeStruct((NW, NSEG, D), jnp.float32),
        mesh=mesh,
        scratch_types=[
            pltpu.VMEM((BLK,), jnp.int32),
            pltpu.VMEM((BLK,), jnp.int32),
            pltpu.VMEM((BLK, D), jnp.float32),
            pltpu.VMEM((BLK, D), jnp.float32),
            pltpu.VMEM((NSEG, D), jnp.float32),
            pltpu.SemaphoreType.DMA,
            pltpu.SemaphoreType.DMA,
        ],
    )
    def k(x_hbm, b_hbm, part_hbm, ids_a, ids_b, buf_a, buf_b, acc_v,
          sem_a, sem_b):
        wid = lax.axis_index("s") * NC + lax.axis_index("c")
        base = wid * CHUNK

        def blk_start(idx):
            return jnp.minimum(base + idx * BLK, LAST_START)

        neg = jnp.full((16,), NEG_INF, jnp.float32)

        def init_body(i, c):
            for f in range(NF):
                acc_v[i, pl.ds(16 * f, 16)] = neg
            return c
        lax.fori_loop(0, NSEG, init_body, 0)

        bufs = ((ids_a, buf_a, sem_a), (ids_b, buf_b, sem_b))

        def issue(idx, ids_v, buf_v, sem):
            s = blk_start(idx)
            pltpu.async_copy(b_hbm.at[pl.ds(s, BLK)], ids_v, sem)
            pltpu.async_copy(x_hbm.at[pl.ds(s, BLK)], buf_v, sem)

        def drain(idx, ids_v, buf_v, sem):
            s = blk_start(idx)
            pltpu.make_async_copy(b_hbm.at[pl.ds(s, BLK)], ids_v, sem).wait()
            pltpu.make_async_copy(x_hbm.at[pl.ds(s, BLK)], buf_v, sem).wait()

        issue(0, *bufs[0])

        def flush(seg, vacc):
            for f in range(NF):
                sl = pl.ds(16 * f, 16)
                acc_v[seg, sl] = jnp.maximum(acc_v[seg, sl], vacc[f])

        def pair_body(p, carry):
            new = carry
            for b in range(2):
                idx = 2 * p + b
                ids_v, buf_v, sem = bufs[b]

                @pl.when(idx + 1 < NBLK)
                def _():
                    issue(idx + 1, *bufs[1 - b])

                drain(idx, ids_v, buf_v, sem)

                def grp_body(t, gc, ids_v=ids_v, buf_v=buf_v):
                    cur_seg, vacc = gc[0], list(gc[1])
                    idv = ids_v[pl.ds(t * 16, 16)]
                    for j in range(16):
                        seg = idv[j]
                        r = t * 16 + j
                        changed = seg != cur_seg

                        @pl.when(changed)
                        def _(cs=cur_seg, va=tuple(vacc)):
                            flush(cs, va)

                        row = [buf_v[r, pl.ds(16 * f, 16)]
                               for f in range(NF)]
                        vacc = [jnp.where(changed, row[f],
                                          jnp.maximum(vacc[f], row[f]))
                                for f in range(NF)]
                        cur_seg = seg
                    return (cur_seg, tuple(vacc))

                new = lax.fori_loop(0, BLK // 16, grp_body, new)
            return new

        init_carry = (jnp.int32(0), tuple(neg for _ in range(NF)))
        final = lax.fori_loop(0, NBLK // 2, pair_body, init_carry)

        flush(final[0], final[1])

        pltpu.sync_copy(acc_v, part_hbm.at[wid])

    return k(x, batch_i32)


def _tc_combine(part):
    def body(p_ref, o_ref):
        o_ref[...] = jnp.max(p_ref[...], axis=0)

    return pl.pallas_call(
        body,
        out_shape=jax.ShapeDtypeStruct((NSEG, D), jnp.float32),
    )(part)


def kernel(x, batch):
    part = _sc_partial_max(x, batch.astype(jnp.int32))
    return _tc_combine(part)
